# baseline (device time: 58934 ns/iter reference)
import jax
import jax.numpy as jnp
from jax import lax
from jax.experimental import pallas as pl
from jax.experimental.pallas import tpu as pltpu

N_DEV = 4


def kernel(x, Win0, Wout0, Win1, Wout1, Win2, Wout2):
    m_per, d = x.shape
    h_per = Win0.shape[1]
    M = N_DEV * m_per

    def body(x_ref, win0, wout0, win1, wout1, win2, wout2, out_ref,
             xfull, partial, comm, send_sems, recv_sems):
        my = lax.axis_index("i")
        right = lax.rem(my + 1, N_DEV)
        left = lax.rem(my + N_DEV - 1, N_DEV)

        barrier_sem = pltpu.get_barrier_semaphore()
        for nbr in (left, right):
            pl.semaphore_signal(
                barrier_sem, inc=1,
                device_id=(nbr,), device_id_type=pl.DeviceIdType.MESH,
            )
        pl.semaphore_wait(barrier_sem, 2)

        wins = (win0, win1, win2)
        wouts = (wout0, wout1, wout2)

        xcur = x_ref[...].astype(jnp.bfloat16)
        ghop = 0

        for l in range(3):
            xfull[pl.ds(my * m_per, m_per), :] = xcur
            comm[ghop % 2] = xcur
            for h in range(N_DEV - 1):
                ss = ghop % 2
                rr = (ghop + 1) % 2
                rdma = pltpu.make_async_remote_copy(
                    src_ref=comm.at[ss],
                    dst_ref=comm.at[rr],
                    send_sem=send_sems.at[ss],
                    recv_sem=recv_sems.at[rr],
                    device_id=(right,),
                    device_id_type=pl.DeviceIdType.MESH,
                )
                rdma.start()
                rdma.wait()
                origin = lax.rem(my - (h + 1) + N_DEV, N_DEV)
                xfull[pl.ds(origin * m_per, m_per), :] = comm[rr]
                ghop += 1

            hpre = jnp.dot(
                xfull[...], wins[l][...].astype(jnp.bfloat16),
                preferred_element_type=jnp.float32,
            )
            hact = jnp.maximum(hpre, 0.0).astype(jnp.bfloat16)
            partial[...] = jnp.dot(
                hact, wouts[l][...].astype(jnp.bfloat16),
                preferred_element_type=jnp.float32,
            )

            for s in range(N_DEV - 1):
                ss = ghop % 2
                rr = (ghop + 1) % 2
                c = lax.rem(my - (s + 1) + N_DEV, N_DEV)
                chunk = partial[pl.ds(c * m_per, m_per), :]
                if s == 0:
                    comm[ss] = chunk.astype(jnp.bfloat16)
                else:
                    comm[ss] = (chunk + comm[ss].astype(jnp.float32)).astype(
                        jnp.bfloat16
                    )
                rdma = pltpu.make_async_remote_copy(
                    src_ref=comm.at[ss],
                    dst_ref=comm.at[rr],
                    send_sem=send_sems.at[ss],
                    recv_sem=recv_sems.at[rr],
                    device_id=(right,),
                    device_id_type=pl.DeviceIdType.MESH,
                )
                rdma.start()
                rdma.wait()
                ghop += 1

            res = (
                partial[pl.ds(my * m_per, m_per), :]
                + comm[ghop % 2].astype(jnp.float32)
            )
            if l < 2:
                xcur = res.astype(jnp.bfloat16)
            else:
                out_ref[...] = res

    return pl.pallas_call(
        body,
        out_shape=jax.ShapeDtypeStruct((m_per, d), jnp.float32),
        in_specs=[pl.BlockSpec(memory_space=pltpu.VMEM)] * 7,
        out_specs=pl.BlockSpec(memory_space=pltpu.VMEM),
        scratch_shapes=[
            pltpu.VMEM((M, d), jnp.bfloat16),
            pltpu.VMEM((M, d), jnp.float32),
            pltpu.VMEM((2, m_per, d), jnp.bfloat16),
            pltpu.SemaphoreType.DMA((2,)),
            pltpu.SemaphoreType.DMA((2,)),
        ],
        compiler_params=pltpu.CompilerParams(collective_id=0),
    )(x, Win0, Wout0, Win1, Wout1, Win2, Wout2)


# device time: 37590 ns/iter; 1.5678x vs baseline; 1.5678x over previous
import jax
import jax.numpy as jnp
from jax import lax
from jax.experimental import pallas as pl
from jax.experimental.pallas import tpu as pltpu

N_DEV = 4


def kernel(x, Win0, Wout0, Win1, Wout1, Win2, Wout2):
    m_per, d = x.shape
    M = N_DEV * m_per

    def body(x_ref, win0, wout0, win1, wout1, win2, wout2, out_ref,
             xfull, partial_bf, rs_recv,
             ag_send_sems, ag_recv_sems, rs_send_sems, rs_recv_sems):
        my = lax.axis_index("i")

        barrier_sem = pltpu.get_barrier_semaphore()
        for delta in range(1, N_DEV):
            pl.semaphore_signal(
                barrier_sem, inc=1,
                device_id=(lax.rem(my + delta, N_DEV),),
                device_id_type=pl.DeviceIdType.MESH,
            )
        pl.semaphore_wait(barrier_sem, N_DEV - 1)

        wins = (win0, win1, win2)
        wouts = (wout0, wout1, wout2)

        my_rows = pl.ds(my * m_per, m_per)
        xfull[my_rows, :] = x_ref[...].astype(jnp.bfloat16)

        for l in range(3):
            ag_sends = []
            for delta in range(1, N_DEV):
                tgt = lax.rem(my + delta, N_DEV)
                rdma = pltpu.make_async_remote_copy(
                    src_ref=xfull.at[my_rows, :],
                    dst_ref=xfull.at[my_rows, :],
                    send_sem=ag_send_sems.at[delta - 1],
                    recv_sem=ag_recv_sems.at[delta - 1],
                    device_id=(tgt,),
                    device_id_type=pl.DeviceIdType.MESH,
                )
                rdma.start()
                ag_sends.append(rdma)
            for delta in range(1, N_DEV):
                src_dev = lax.rem(my - delta + N_DEV, N_DEV)
                recv = pltpu.make_async_remote_copy(
                    src_ref=xfull.at[my_rows, :],
                    dst_ref=xfull.at[pl.ds(src_dev * m_per, m_per), :],
                    send_sem=ag_send_sems.at[delta - 1],
                    recv_sem=ag_recv_sems.at[delta - 1],
                    device_id=(my,),
                    device_id_type=pl.DeviceIdType.MESH,
                )
                recv.wait_recv()
            for rdma in ag_sends:
                rdma.wait_send()

            hact = jnp.maximum(
                jnp.dot(
                    xfull[...], wins[l][...].astype(jnp.bfloat16),
                    preferred_element_type=jnp.float32,
                ),
                0.0,
            ).astype(jnp.bfloat16)
            partial_bf[...] = jnp.dot(
                hact, wouts[l][...].astype(jnp.bfloat16),
                preferred_element_type=jnp.float32,
            ).astype(jnp.bfloat16)

            rs_sends = []
            for delta in range(1, N_DEV):
                tgt = lax.rem(my + delta, N_DEV)
                rdma = pltpu.make_async_remote_copy(
                    src_ref=partial_bf.at[pl.ds(tgt * m_per, m_per), :],
                    dst_ref=rs_recv.at[delta - 1],
                    send_sem=rs_send_sems.at[delta - 1],
                    recv_sem=rs_recv_sems.at[delta - 1],
                    device_id=(tgt,),
                    device_id_type=pl.DeviceIdType.MESH,
                )
                rdma.start()
                rs_sends.append(rdma)
            for delta in range(1, N_DEV):
                recv = pltpu.make_async_remote_copy(
                    src_ref=partial_bf.at[my_rows, :],
                    dst_ref=rs_recv.at[delta - 1],
                    send_sem=rs_send_sems.at[delta - 1],
                    recv_sem=rs_recv_sems.at[delta - 1],
                    device_id=(my,),
                    device_id_type=pl.DeviceIdType.MESH,
                )
                recv.wait_recv()

            res = partial_bf[my_rows, :].astype(jnp.float32)
            for j in range(N_DEV - 1):
                res = res + rs_recv[j].astype(jnp.float32)
            for rdma in rs_sends:
                rdma.wait_send()

            if l < 2:
                xfull[my_rows, :] = res.astype(jnp.bfloat16)
            else:
                out_ref[...] = res

    return pl.pallas_call(
        body,
        out_shape=jax.ShapeDtypeStruct((m_per, d), jnp.float32),
        in_specs=[pl.BlockSpec(memory_space=pltpu.VMEM)] * 7,
        out_specs=pl.BlockSpec(memory_space=pltpu.VMEM),
        scratch_shapes=[
            pltpu.VMEM((M, d), jnp.bfloat16),
            pltpu.VMEM((M, d), jnp.bfloat16),
            pltpu.VMEM((N_DEV - 1, m_per, d), jnp.bfloat16),
            pltpu.SemaphoreType.DMA((N_DEV - 1,)),
            pltpu.SemaphoreType.DMA((N_DEV - 1,)),
            pltpu.SemaphoreType.DMA((N_DEV - 1,)),
            pltpu.SemaphoreType.DMA((N_DEV - 1,)),
        ],
        compiler_params=pltpu.CompilerParams(collective_id=0),
    )(x, Win0, Wout0, Win1, Wout1, Win2, Wout2)


# device time: 37413 ns/iter; 1.5752x vs baseline; 1.0047x over previous
import jax
import jax.numpy as jnp
from jax import lax
from jax.experimental import pallas as pl
from jax.experimental.pallas import tpu as pltpu

N_DEV = 4


def kernel(x, Win0, Wout0, Win1, Wout1, Win2, Wout2):
    m_per, d = x.shape
    M = N_DEV * m_per

    def body(x_ref, win0, wout0, win1, wout1, win2, wout2, out_ref,
             xfull, partial_bf, rs_recv,
             ag_send_sems, ag_recv_sems, rs_send_sems, rs_recv_sems):
        my = lax.axis_index("i")

        barrier_sem = pltpu.get_barrier_semaphore()
        for delta in range(1, N_DEV):
            pl.semaphore_signal(
                barrier_sem, inc=1,
                device_id=(lax.rem(my + delta, N_DEV),),
                device_id_type=pl.DeviceIdType.MESH,
            )
        pl.semaphore_wait(barrier_sem, N_DEV - 1)

        wins = (win0, win1, win2)
        wouts = (wout0, wout1, wout2)

        my_rows = pl.ds(my * m_per, m_per)
        xfull[my_rows, :] = x_ref[...].astype(jnp.bfloat16)

        def chunk_partial(rows, wl, wo):
            h = jnp.dot(xfull[rows, :], wl, preferred_element_type=jnp.float32)
            h = jnp.maximum(h, 0.0).astype(jnp.bfloat16)
            return jnp.dot(h, wo, preferred_element_type=jnp.float32)

        for l in range(3):
            wl = wins[l][...].astype(jnp.bfloat16)
            wo = wouts[l][...].astype(jnp.bfloat16)

            ag_sends = []
            for delta in range(1, N_DEV):
                tgt = lax.rem(my + delta, N_DEV)
                rdma = pltpu.make_async_remote_copy(
                    src_ref=xfull.at[my_rows, :],
                    dst_ref=xfull.at[my_rows, :],
                    send_sem=ag_send_sems.at[delta - 1],
                    recv_sem=ag_recv_sems.at[delta - 1],
                    device_id=(tgt,),
                    device_id_type=pl.DeviceIdType.MESH,
                )
                rdma.start()
                ag_sends.append(rdma)

            own = chunk_partial(my_rows, wl, wo)

            rs_sends = []
            for delta in (1, 3, 2):
                src_dev = lax.rem(my - delta + N_DEV, N_DEV)
                c_rows = pl.ds(src_dev * m_per, m_per)
                recv = pltpu.make_async_remote_copy(
                    src_ref=xfull.at[my_rows, :],
                    dst_ref=xfull.at[c_rows, :],
                    send_sem=ag_send_sems.at[delta - 1],
                    recv_sem=ag_recv_sems.at[delta - 1],
                    device_id=(my,),
                    device_id_type=pl.DeviceIdType.MESH,
                )
                recv.wait_recv()
                partial_bf[c_rows, :] = chunk_partial(c_rows, wl, wo).astype(
                    jnp.bfloat16
                )
                slot = (N_DEV - delta) - 1
                rdma = pltpu.make_async_remote_copy(
                    src_ref=partial_bf.at[c_rows, :],
                    dst_ref=rs_recv.at[slot],
                    send_sem=rs_send_sems.at[slot],
                    recv_sem=rs_recv_sems.at[slot],
                    device_id=(src_dev,),
                    device_id_type=pl.DeviceIdType.MESH,
                )
                rdma.start()
                rs_sends.append(rdma)
            for rdma in ag_sends:
                rdma.wait_send()

            for slot in range(N_DEV - 1):
                recv = pltpu.make_async_remote_copy(
                    src_ref=partial_bf.at[my_rows, :],
                    dst_ref=rs_recv.at[slot],
                    send_sem=rs_send_sems.at[slot],
                    recv_sem=rs_recv_sems.at[slot],
                    device_id=(my,),
                    device_id_type=pl.DeviceIdType.MESH,
                )
                recv.wait_recv()

            res = own
            for j in range(N_DEV - 1):
                res = res + rs_recv[j].astype(jnp.float32)
            for rdma in rs_sends:
                rdma.wait_send()

            if l < 2:
                xfull[my_rows, :] = res.astype(jnp.bfloat16)
            else:
                out_ref[...] = res

    return pl.pallas_call(
        body,
        out_shape=jax.ShapeDtypeStruct((m_per, d), jnp.float32),
        in_specs=[pl.BlockSpec(memory_space=pltpu.VMEM)] * 7,
        out_specs=pl.BlockSpec(memory_space=pltpu.VMEM),
        scratch_shapes=[
            pltpu.VMEM((M, d), jnp.bfloat16),
            pltpu.VMEM((M, d), jnp.bfloat16),
            pltpu.VMEM((N_DEV - 1, m_per, d), jnp.bfloat16),
            pltpu.SemaphoreType.DMA((N_DEV - 1,)),
            pltpu.SemaphoreType.DMA((N_DEV - 1,)),
            pltpu.SemaphoreType.DMA((N_DEV - 1,)),
            pltpu.SemaphoreType.DMA((N_DEV - 1,)),
        ],
        compiler_params=pltpu.CompilerParams(collective_id=0),
    )(x, Win0, Wout0, Win1, Wout1, Win2, Wout2)


# device time: 36098 ns/iter; 1.6326x vs baseline; 1.0364x over previous
import jax
import jax.numpy as jnp
from jax import lax
from jax.experimental import pallas as pl
from jax.experimental.pallas import tpu as pltpu

N_DEV = 4


def kernel(x, Win0, Wout0, Win1, Wout1, Win2, Wout2):
    m_per, d = x.shape
    M = N_DEV * m_per

    def body(x_ref, win0, wout0, win1, wout1, win2, wout2, out_ref,
             xfull, partial_bf, rs_recv,
             ag_send_sems, ag_recv_sems, rs_send_sems, rs_recv_sems):
        my = lax.axis_index("i")
        my_rows = pl.ds(my * m_per, m_per)
        xfull[my_rows, :] = x_ref[...].astype(jnp.bfloat16)

        barrier_sem = pltpu.get_barrier_semaphore()
        for delta in range(1, N_DEV):
            pl.semaphore_signal(
                barrier_sem, inc=1,
                device_id=(lax.rem(my + delta, N_DEV),),
                device_id_type=pl.DeviceIdType.MESH,
            )
        pl.semaphore_wait(barrier_sem, N_DEV - 1)

        wins = (win0, win1, win2)
        wouts = (wout0, wout1, wout2)

        def chunk_partial(rows, wl, wo):
            h = jnp.dot(xfull[rows, :], wl, preferred_element_type=jnp.float32)
            h = jnp.maximum(h, 0.0).astype(jnp.bfloat16)
            return jnp.dot(h, wo, preferred_element_type=jnp.float32)

        for l in range(3):
            wl = wins[l][...]
            wo = wouts[l][...]

            ag_sends = []
            for delta in range(1, N_DEV):
                tgt = lax.rem(my + delta, N_DEV)
                rdma = pltpu.make_async_remote_copy(
                    src_ref=xfull.at[my_rows, :],
                    dst_ref=xfull.at[my_rows, :],
                    send_sem=ag_send_sems.at[delta - 1],
                    recv_sem=ag_recv_sems.at[delta - 1],
                    device_id=(tgt,),
                    device_id_type=pl.DeviceIdType.MESH,
                )
                rdma.start()
                ag_sends.append(rdma)

            own = chunk_partial(my_rows, wl, wo)

            rs_sends = []
            for delta in (1, 3, 2):
                src_dev = lax.rem(my - delta + N_DEV, N_DEV)
                c_rows = pl.ds(src_dev * m_per, m_per)
                recv = pltpu.make_async_remote_copy(
                    src_ref=xfull.at[my_rows, :],
                    dst_ref=xfull.at[c_rows, :],
                    send_sem=ag_send_sems.at[delta - 1],
                    recv_sem=ag_recv_sems.at[delta - 1],
                    device_id=(my,),
                    device_id_type=pl.DeviceIdType.MESH,
                )
                recv.wait_recv()
                partial_bf[c_rows, :] = chunk_partial(c_rows, wl, wo).astype(
                    jnp.bfloat16
                )
                slot = (N_DEV - delta) - 1
                rdma = pltpu.make_async_remote_copy(
                    src_ref=partial_bf.at[c_rows, :],
                    dst_ref=rs_recv.at[slot],
                    send_sem=rs_send_sems.at[slot],
                    recv_sem=rs_recv_sems.at[slot],
                    device_id=(src_dev,),
                    device_id_type=pl.DeviceIdType.MESH,
                )
                rdma.start()
                rs_sends.append(rdma)
            for rdma in ag_sends:
                rdma.wait_send()

            for slot in range(N_DEV - 1):
                recv = pltpu.make_async_remote_copy(
                    src_ref=partial_bf.at[my_rows, :],
                    dst_ref=rs_recv.at[slot],
                    send_sem=rs_send_sems.at[slot],
                    recv_sem=rs_recv_sems.at[slot],
                    device_id=(my,),
                    device_id_type=pl.DeviceIdType.MESH,
                )
                recv.wait_recv()

            res = own
            for j in range(N_DEV - 1):
                res = res + rs_recv[j].astype(jnp.float32)
            for rdma in rs_sends:
                rdma.wait_send()

            if l < 2:
                xfull[my_rows, :] = res.astype(jnp.bfloat16)
            else:
                out_ref[...] = res

    return pl.pallas_call(
        body,
        out_shape=jax.ShapeDtypeStruct((m_per, d), jnp.float32),
        in_specs=[pl.BlockSpec(memory_space=pltpu.VMEM)] * 7,
        out_specs=pl.BlockSpec(memory_space=pltpu.VMEM),
        scratch_shapes=[
            pltpu.VMEM((M, d), jnp.bfloat16),
            pltpu.VMEM((M, d), jnp.bfloat16),
            pltpu.VMEM((N_DEV - 1, m_per, d), jnp.bfloat16),
            pltpu.SemaphoreType.DMA((N_DEV - 1,)),
            pltpu.SemaphoreType.DMA((N_DEV - 1,)),
            pltpu.SemaphoreType.DMA((N_DEV - 1,)),
            pltpu.SemaphoreType.DMA((N_DEV - 1,)),
        ],
        compiler_params=pltpu.CompilerParams(collective_id=0),
    )(
        x,
        Win0.astype(jnp.bfloat16), Wout0.astype(jnp.bfloat16),
        Win1.astype(jnp.bfloat16), Wout1.astype(jnp.bfloat16),
        Win2.astype(jnp.bfloat16), Wout2.astype(jnp.bfloat16),
    )
